# speedup vs baseline: 1.0006x; 1.0006x over previous
import functools

import jax
import jax.numpy as jnp
from jax.experimental import pallas as pl
from jax.experimental.pallas import tpu as pltpu

NUM_USERS = 100000
NUM_ITEMS = 100000
EMB = 128
E = 625000

ROW_BLK = 2000


def _proj_body(u_ref, i_ref, wq_ref, wk_ref, wv_ref, uq_ref, ik_ref, iv_ref):
    u = u_ref[...]
    it = i_ref[...]
    dn = (((1,), (1,)), ((), ()))
    uq_ref[...] = jax.lax.dot_general(u, wq_ref[...], dn,
                                      preferred_element_type=jnp.float32)
    ik_ref[...] = jax.lax.dot_general(it, wk_ref[...], dn,
                                      preferred_element_type=jnp.float32)
    iv_ref[...] = jax.lax.dot_general(it, wv_ref[...], dn,
                                      preferred_element_type=jnp.float32)


@jax.jit
def _projections(user_embed, item_embed, Wq, Wk, Wv):
    n_blk = NUM_USERS // ROW_BLK
    row_spec = pl.BlockSpec((ROW_BLK, EMB), lambda i: (i, 0))
    w_spec = pl.BlockSpec((EMB, EMB), lambda i: (0, 0))
    out_shape = jax.ShapeDtypeStruct((NUM_USERS, EMB), jnp.float32)
    return pl.pallas_call(
        _proj_body,
        grid=(n_blk,),
        in_specs=[row_spec, row_spec, w_spec, w_spec, w_spec],
        out_specs=[row_spec, row_spec, row_spec],
        out_shape=[out_shape, out_shape, out_shape],
    )(user_embed, item_embed, Wq, Wk, Wv)


@jax.jit
def kernel(inter_edge, user_embed, item_embed, Wq, Wk, Wv):
    users = inter_edge[0, :]
    items = inter_edge[1, :]
    uq, ik, iv = _projections(user_embed, item_embed, Wq, Wk, Wv)
    query = uq[users]
    key = ik[items]
    value = iv[items]
    scores = (query * key).sum(axis=-1) / (EMB ** 0.5)
    seg_max = jax.ops.segment_max(scores, users, num_segments=NUM_USERS)
    seg_max = jnp.where(jnp.isfinite(seg_max), seg_max, 0.0)
    ex = jnp.exp(scores - seg_max[users])
    denom = jax.ops.segment_sum(ex, users, num_segments=NUM_USERS)
    alpha = ex / (denom[users] + 1e-16)
    weighted_value = value * alpha[:, None]
    sums = jax.ops.segment_sum(weighted_value, users, num_segments=NUM_USERS)
    counts = jax.ops.segment_sum(jnp.ones_like(scores), users,
                                 num_segments=NUM_USERS)
    return sums / jnp.clip(counts, 1.0, None)[:, None]


# speedup vs baseline: 3.1816x; 3.1795x over previous
import functools

import jax
import jax.numpy as jnp
from jax import lax
from jax.experimental import pallas as pl
from jax.experimental.pallas import tpu as pltpu
from jax.experimental.pallas import tpu_sc as plsc

NUM_USERS = 100000
NUM_ITEMS = 100000
EMB = 128
E = 625000

NC = 2
NS = 16
NW = NC * NS
L = 16

B_PER_W = 19968
E_PAD = B_PER_W * NW
U_PAD = 102400
CHK = 128
W3 = 512
EBLK = 2048
NBLK = E_PAD // EBLK
CG = EMB // L

_mesh = functools.partial(plsc.VectorSubcoreMesh,
                          core_axis_name="c", subcore_axis_name="s")


def _wid():
    return lax.axis_index("s") * NC + lax.axis_index("c")


def _s0_body(users_hbm, items_hbm, ue_hbm, ie_hbm, ug_hbm, ig_hbm,
             uidx, iidx, urows, irows, sem):
    wid = _wid()
    base = wid * B_PER_W

    def win(w, carry):
        off = base + w * CHK
        pltpu.sync_copy(users_hbm.at[pl.ds(off, CHK)], uidx)
        pltpu.sync_copy(items_hbm.at[pl.ds(off, CHK)], iidx)
        cu = pltpu.async_copy(ue_hbm.at[uidx], urows, sem)
        cu.wait()
        pltpu.sync_copy(urows, ug_hbm.at[pl.ds(off, CHK)])
        ci = pltpu.async_copy(ie_hbm.at[iidx], irows, sem)
        ci.wait()
        pltpu.sync_copy(irows, ig_hbm.at[pl.ds(off, CHK)])
        return carry

    lax.fori_loop(0, B_PER_W // CHK, win, 0)


def _s0(users_p, items_p, user_embed, item_embed):
    return pl.kernel(
        _s0_body,
        out_type=(
            jax.ShapeDtypeStruct((E_PAD, EMB), jnp.float32),
            jax.ShapeDtypeStruct((E_PAD, EMB), jnp.float32),
        ),
        mesh=_mesh(),
        scratch_types=[
            pltpu.VMEM((CHK,), jnp.int32),
            pltpu.VMEM((CHK,), jnp.int32),
            pltpu.VMEM((CHK, EMB), jnp.float32),
            pltpu.VMEM((CHK, EMB), jnp.float32),
            pltpu.SemaphoreType.DMA,
        ],
    )(users_p, items_p, user_embed, item_embed)


def _t2_body(ug_ref, ig_ref, wq_ref, wk_ref, s_ref, bm_ref):
    m = lax.dot_general(wq_ref[...], wk_ref[...], (((0,), (0,)), ((), ())),
                        preferred_element_type=jnp.float32)
    p = lax.dot_general(ug_ref[...], m, (((1,), (0,)), ((), ())),
                        preferred_element_type=jnp.float32)
    s = (p * ig_ref[...]).sum(axis=1) * jnp.float32(1.0 / (EMB ** 0.5))
    s_ref[...] = s[None, None, :]
    bm_ref[...] = jnp.full((1, 1, 128), jnp.max(s), jnp.float32)


def _t2(ug, ig, Wq, Wk):
    eb = pl.BlockSpec((EBLK, EMB), lambda i: (i, 0))
    w_spec = pl.BlockSpec((EMB, EMB), lambda i: (0, 0))
    return pl.pallas_call(
        _t2_body,
        grid=(NBLK,),
        in_specs=[eb, eb, w_spec, w_spec],
        out_specs=[pl.BlockSpec((1, 1, EBLK), lambda i: (i, 0, 0)),
                   pl.BlockSpec((1, 1, 128), lambda i: (i, 0, 0))],
        out_shape=[jax.ShapeDtypeStruct((NBLK, 1, EBLK), jnp.float32),
                   jax.ShapeDtypeStruct((NBLK, 1, 128), jnp.float32)],
    )(ug, ig, Wq, Wk)


def _s2_body(users_hbm, scores_hbm, gmax_hbm, den_hbm, cnt_hbm,
             ubuf, sbuf, gbuf, exbuf, onebuf, zbuf, den_sh, cnt_sh):
    wid = _wid()
    cid = lax.axis_index("c")
    sid = lax.axis_index("s")
    base = wid * B_PER_W
    slc = U_PAD // NS

    def z(i, c):
        zbuf[pl.ds(i * L, L)] = jnp.zeros((L,), jnp.float32)
        return c
    lax.fori_loop(0, slc // L, z, 0)
    pltpu.sync_copy(zbuf, den_sh.at[pl.ds(sid * slc, slc)])
    pltpu.sync_copy(zbuf, cnt_sh.at[pl.ds(sid * slc, slc)])
    pltpu.sync_copy(gmax_hbm, gbuf)
    plsc.subcore_barrier()
    g = gbuf[...]

    def win(w, carry):
        off = base + w * CHK
        pltpu.sync_copy(users_hbm.at[pl.ds(off, CHK)], ubuf.at[0])
        pltpu.sync_copy(scores_hbm.at[pl.ds(off, CHK)], sbuf)

        def vec(i, c2):
            s = sbuf[pl.ds(i * L, L)]
            valid = (off + i * L + jax.lax.iota(jnp.int32, L)) < E
            ex = jnp.where(valid, jnp.exp(s - g), 0.0)
            one = jnp.where(valid, 1.0, 0.0).astype(jnp.float32)
            exbuf[0, pl.ds(i * L, L)] = ex
            onebuf[0, pl.ds(i * L, L)] = one
            return c2
        lax.fori_loop(0, CHK // L, vec, 0)
        pltpu.sync_copy(exbuf.at[0], den_sh.at[ubuf.at[0]], add=True)
        pltpu.sync_copy(onebuf.at[0], cnt_sh.at[ubuf.at[0]], add=True)
        return carry

    lax.fori_loop(0, B_PER_W // CHK, win, 0)
    plsc.subcore_barrier()
    pltpu.sync_copy(den_sh.at[pl.ds(sid * slc, slc)],
                    den_hbm.at[cid, pl.ds(sid * slc, slc)])
    pltpu.sync_copy(cnt_sh.at[pl.ds(sid * slc, slc)],
                    cnt_hbm.at[cid, pl.ds(sid * slc, slc)])


def _s2(users_p, scores, gmax_arr):
    return pl.kernel(
        _s2_body,
        out_type=(
            jax.ShapeDtypeStruct((NC, U_PAD), jnp.float32),
            jax.ShapeDtypeStruct((NC, U_PAD), jnp.float32),
        ),
        mesh=_mesh(),
        scratch_types=[
            pltpu.VMEM((1, CHK), jnp.int32),
            pltpu.VMEM((CHK,), jnp.float32),
            pltpu.VMEM((L,), jnp.float32),
            pltpu.VMEM((1, CHK), jnp.float32),
            pltpu.VMEM((1, CHK), jnp.float32),
            pltpu.VMEM((U_PAD // NS,), jnp.float32),
            pltpu.VMEM_SHARED((U_PAD,), jnp.float32),
            pltpu.VMEM_SHARED((U_PAD,), jnp.float32),
        ],
    )(users_p, scores, gmax_arr)


def _t3_body(s_ref, gm_ref, ig_ref, wg_ref):
    ex = jnp.exp(s_ref[0, 0, :] - gm_ref[0, 0])
    pos = pl.program_id(0) * EBLK + lax.broadcasted_iota(jnp.int32, (EBLK,), 0)
    ex = jnp.where(pos < E, ex, 0.0)
    wg = ig_ref[...] * ex[:, None]
    wg_ref[...] = wg.reshape(EBLK, CG, L).transpose(1, 0, 2)


def _t3(scores3d, gmax2d, ig):
    return pl.pallas_call(
        _t3_body,
        grid=(NBLK,),
        in_specs=[pl.BlockSpec((1, 1, EBLK), lambda i: (i, 0, 0)),
                  pl.BlockSpec((1, 128), lambda i: (0, 0)),
                  pl.BlockSpec((EBLK, EMB), lambda i: (i, 0))],
        out_specs=pl.BlockSpec((CG, EBLK, L), lambda i: (0, i, 0)),
        out_shape=jax.ShapeDtypeStruct((CG, E_PAD, L), jnp.float32),
    )(scores3d, gmax2d, ig)


UHALF = U_PAD // NC
DUMP = 256
ACC_R = UHALF + DUMP


def _s3_body(users_hbm, wg_hbm, raw_hbm, ufl, ubuf, wbuf, zbuf, acc_sh):
    cid = lax.axis_index("c")
    sid = lax.axis_index("s")
    base = sid * (E_PAD // NS)
    slc = UHALF // NS
    zslc = ACC_R // NS
    lo = cid * UHALF
    lane = jax.lax.iota(jnp.int32, L)

    def zi(i, c):
        zbuf[i, pl.ds(0, L)] = jnp.zeros((L,), jnp.float32)
        return c
    lax.fori_loop(0, 100, zi, 0)

    def rnd(rg, carry):
        def zc(i, c):
            pltpu.sync_copy(zbuf, acc_sh.at[pl.ds(sid * zslc + i * 100, 100)])
            return c
        lax.fori_loop(0, zslc // 100, zc, 0)
        pltpu.sync_copy(zbuf.at[pl.ds(0, 16)],
                        acc_sh.at[pl.ds(sid * zslc + (zslc // 100) * 100, 16)])
        plsc.subcore_barrier()

        def win(w, carry2):
            off = base + w * W3
            pltpu.sync_copy(wg_hbm.at[rg, pl.ds(off, W3)], wbuf)
            for q in range(W3 // CHK):
                pltpu.sync_copy(users_hbm.at[pl.ds(off + q * CHK, CHK)], ufl)

                def vec(i, c3):
                    u = ufl[pl.ds(i * L, L)]
                    inh = (u >= lo) & (u < lo + UHALF)
                    lu = jnp.where(inh, u - lo, UHALF + (u & (DUMP - 1)))
                    ubuf[q, pl.ds(i * L, L)] = lu
                    return c3
                lax.fori_loop(0, CHK // L, vec, 0)
            for q in range(W3 // CHK):
                pltpu.sync_copy(wbuf.at[pl.ds(q * CHK, CHK)],
                                acc_sh.at[ubuf.at[q]], add=True)
            return carry2
        lax.fori_loop(0, (E_PAD // NS) // W3, win, 0)

        plsc.subcore_barrier()
        pltpu.sync_copy(acc_sh.at[pl.ds(sid * slc, slc)],
                        raw_hbm.at[rg, cid, pl.ds(sid * slc, slc)])
        plsc.subcore_barrier()
        return carry

    lax.fori_loop(0, CG, rnd, 0)


def _s3(users_p, wg):
    return pl.kernel(
        _s3_body,
        out_type=jax.ShapeDtypeStruct((CG, NC, UHALF, L), jnp.float32),
        mesh=_mesh(),
        compiler_params=pltpu.CompilerParams(use_tc_tiling_on_sc=False),
        scratch_types=[
            pltpu.VMEM((CHK,), jnp.int32),
            pltpu.VMEM((W3 // CHK, CHK), jnp.int32),
            pltpu.VMEM((W3, L), jnp.float32),
            pltpu.VMEM((100, L), jnp.float32),
            pltpu.VMEM_SHARED((ACC_R, L), jnp.float32),
        ],
    )(users_p, wg)


def _t4_body(raw_ref, d_ref, c_ref, wv_ref, out_ref):
    r = raw_ref[...].reshape(CG, 1024, L)
    r = r.transpose(1, 0, 2).reshape(1024, EMB)
    agg = lax.dot_general(r, wv_ref[...], (((1,), (1,)), ((), ())),
                          preferred_element_type=jnp.float32)
    d = d_ref[0] + d_ref[1] + jnp.float32(1e-16)
    cnt = jnp.maximum(c_ref[0] + c_ref[1], 1.0)
    scale = 1.0 / (d * cnt)
    out_ref[...] = agg.reshape(8, 128, EMB) * scale[:, :, None]


def _t4(raw, den3, cnt3, Wv):
    n_blk = U_PAD // 1024
    return pl.pallas_call(
        _t4_body,
        grid=(n_blk,),
        in_specs=[
            pl.BlockSpec((CG, 1, 1024, L), lambda i: (0, i // 50, i % 50, 0)),
            pl.BlockSpec((NC, 8, 128), lambda i: (0, i, 0)),
            pl.BlockSpec((NC, 8, 128), lambda i: (0, i, 0)),
            pl.BlockSpec((EMB, EMB), lambda i: (0, 0)),
        ],
        out_specs=pl.BlockSpec((8, 128, EMB), lambda i: (i, 0, 0)),
        out_shape=jax.ShapeDtypeStruct((U_PAD // 128, 128, EMB), jnp.float32),
    )(raw, den3, cnt3, Wv)


_BISECT = 9


@jax.jit
def kernel(inter_edge, user_embed, item_embed, Wq, Wk, Wv):
    users = inter_edge[0, :].astype(jnp.int32)
    items = inter_edge[1, :].astype(jnp.int32)
    pad = E_PAD - E
    pad_u = (jnp.arange(pad, dtype=jnp.int32) * 97) % NUM_USERS
    pad_i = (jnp.arange(pad, dtype=jnp.int32) * 89) % NUM_ITEMS
    users_p = jnp.concatenate([users, pad_u])
    items_p = jnp.concatenate([items, pad_i])

    ug, ig = _s0(users_p, items_p, user_embed, item_embed)
    if _BISECT == 0:
        return jnp.full((NUM_USERS, EMB), ug[0, 0] + ig[0, 0])
    scores3d, bmax = _t2(ug, ig, Wq, Wk)
    gmax = jnp.max(bmax)
    scores = scores3d.reshape(E_PAD)
    gmax_arr = jnp.full((L,), gmax, jnp.float32)
    gmax2d = jnp.full((1, 128), gmax, jnp.float32)
    den, cnt = _s2(users_p, scores, gmax_arr)
    if _BISECT == 2:
        return jnp.full((NUM_USERS, EMB), den[0, 0] + cnt[0, 0])
    wg = _t3(scores3d, gmax2d, ig)
    raw = _s3(users_p, wg)
    den3 = den.reshape(NC, U_PAD // 128, 128)
    cnt3 = cnt.reshape(NC, U_PAD // 128, 128)
    out3 = _t4(raw, den3, cnt3, Wv)
    return out3.reshape(U_PAD, EMB)[:NUM_USERS]


# speedup vs baseline: 6.9500x; 2.1845x over previous
import functools

import jax
import jax.numpy as jnp
from jax import lax
from jax.experimental import pallas as pl
from jax.experimental.pallas import tpu as pltpu
from jax.experimental.pallas import tpu_sc as plsc

NUM_USERS = 100000
NUM_ITEMS = 100000
EMB = 128
E = 625000

NC = 2
NS = 16
NW = NC * NS
L = 16

B_PER_W = 19968
E_PAD = B_PER_W * NW
U_PAD = 102400
CHK = 128
W3 = 512
EBLK = 2048
NBLK = E_PAD // EBLK
CG = EMB // L

_mesh = functools.partial(plsc.VectorSubcoreMesh,
                          core_axis_name="c", subcore_axis_name="s")


def _wid():
    return lax.axis_index("s") * NC + lax.axis_index("c")


W0 = 256


def _s0_body(users_hbm, items_hbm, ue_hbm, ie_hbm, ug_hbm, ig_hbm,
             uidx, iidx, urows, irows, sem):
    wid = _wid()
    base = wid * B_PER_W

    def win(w, carry):
        off = base + w * W0
        cs = []
        for q in range(W0 // CHK):
            cs.append(pltpu.async_copy(
                users_hbm.at[pl.ds(off + q * CHK, CHK)], uidx.at[q], sem))
            cs.append(pltpu.async_copy(
                items_hbm.at[pl.ds(off + q * CHK, CHK)], iidx.at[q], sem))
        for c in cs:
            c.wait()
        gs = []
        for q in range(W0 // CHK):
            gs.append(pltpu.async_copy(
                ue_hbm.at[uidx.at[q]],
                urows.at[pl.ds(q * CHK, CHK)], sem))
            gs.append(pltpu.async_copy(
                ie_hbm.at[iidx.at[q]],
                irows.at[pl.ds(q * CHK, CHK)], sem))
        for g in gs:
            g.wait()
        w1 = pltpu.async_copy(urows, ug_hbm.at[pl.ds(off, W0)], sem)
        w2 = pltpu.async_copy(irows, ig_hbm.at[pl.ds(off, W0)], sem)
        w1.wait()
        w2.wait()
        return carry

    lax.fori_loop(0, B_PER_W // W0, win, 0)


def _s0(users_p, items_p, user_embed, item_embed):
    return pl.kernel(
        _s0_body,
        out_type=(
            jax.ShapeDtypeStruct((E_PAD, EMB), jnp.float32),
            jax.ShapeDtypeStruct((E_PAD, EMB), jnp.float32),
        ),
        mesh=_mesh(),
        scratch_types=[
            pltpu.VMEM((W0 // CHK, CHK), jnp.int32),
            pltpu.VMEM((W0 // CHK, CHK), jnp.int32),
            pltpu.VMEM((W0, EMB), jnp.float32),
            pltpu.VMEM((W0, EMB), jnp.float32),
            pltpu.SemaphoreType.DMA,
        ],
    )(users_p, items_p, user_embed, item_embed)


def _t2_body(ug_ref, ig_ref, wq_ref, wk_ref, s_ref, bm_ref):
    m = lax.dot_general(wq_ref[...], wk_ref[...], (((0,), (0,)), ((), ())),
                        preferred_element_type=jnp.float32)
    p = lax.dot_general(ug_ref[...], m, (((1,), (0,)), ((), ())),
                        preferred_element_type=jnp.float32)
    s = (p * ig_ref[...]).sum(axis=1) * jnp.float32(1.0 / (EMB ** 0.5))
    s_ref[...] = s[None, None, :]
    bm_ref[...] = jnp.full((1, 1, 128), jnp.max(s), jnp.float32)


def _t2(ug, ig, Wq, Wk):
    eb = pl.BlockSpec((EBLK, EMB), lambda i: (i, 0))
    w_spec = pl.BlockSpec((EMB, EMB), lambda i: (0, 0))
    return pl.pallas_call(
        _t2_body,
        grid=(NBLK,),
        in_specs=[eb, eb, w_spec, w_spec],
        out_specs=[pl.BlockSpec((1, 1, EBLK), lambda i: (i, 0, 0)),
                   pl.BlockSpec((1, 1, 128), lambda i: (i, 0, 0))],
        out_shape=[jax.ShapeDtypeStruct((NBLK, 1, EBLK), jnp.float32),
                   jax.ShapeDtypeStruct((NBLK, 1, 128), jnp.float32)],
    )(ug, ig, Wq, Wk)


def _s2_body(users_hbm, scores_hbm, gmax_hbm, ig_hbm,
             den_hbm, cnt_hbm, wg_hbm,
             ubuf, sbuf, gbuf, exbuf, onebuf, zbuf, igbuf, sem,
             den_sh, cnt_sh):
    wid = _wid()
    cid = lax.axis_index("c")
    sid = lax.axis_index("s")
    base = wid * B_PER_W
    slc = U_PAD // NS

    def z(i, c):
        zbuf[pl.ds(i * L, L)] = jnp.zeros((L,), jnp.float32)
        return c
    lax.fori_loop(0, slc // L, z, 0)
    pltpu.sync_copy(zbuf, den_sh.at[pl.ds(sid * slc, slc)])
    pltpu.sync_copy(zbuf, cnt_sh.at[pl.ds(sid * slc, slc)])
    pltpu.sync_copy(gmax_hbm, gbuf)
    plsc.subcore_barrier()
    g = gbuf[...]

    def win(w, carry):
        off = base + w * CHK
        c1 = pltpu.async_copy(users_hbm.at[pl.ds(off, CHK)], ubuf.at[0], sem)
        c2 = pltpu.async_copy(scores_hbm.at[pl.ds(off, CHK)], sbuf, sem)
        c3 = pltpu.async_copy(ig_hbm.at[pl.ds(off, CHK)], igbuf, sem)
        c1.wait()
        c2.wait()
        c3.wait()

        def vec(i, c2_):
            s = sbuf[pl.ds(i * L, L)]
            valid = (off + i * L + jax.lax.iota(jnp.int32, L)) < E
            ex = jnp.where(valid, jnp.exp(s - g), 0.0)
            one = jnp.where(valid, 1.0, 0.0).astype(jnp.float32)
            exbuf[0, pl.ds(i * L, L)] = ex
            onebuf[0, pl.ds(i * L, L)] = one
            for r16 in range(L):
                r = i * L + r16
                b = jnp.full((L,), ex[r16], jnp.float32)
                for j in range(EMB // L):
                    igbuf[r, pl.ds(j * L, L)] = igbuf[r, pl.ds(j * L, L)] * b
            return c2_
        lax.fori_loop(0, CHK // L, vec, 0)
        w1 = pltpu.async_copy(igbuf, wg_hbm.at[pl.ds(off, CHK)], sem)
        pltpu.sync_copy(exbuf.at[0], den_sh.at[ubuf.at[0]], add=True)
        pltpu.sync_copy(onebuf.at[0], cnt_sh.at[ubuf.at[0]], add=True)
        w1.wait()
        return carry

    lax.fori_loop(0, B_PER_W // CHK, win, 0)
    plsc.subcore_barrier()
    pltpu.sync_copy(den_sh.at[pl.ds(sid * slc, slc)],
                    den_hbm.at[cid, pl.ds(sid * slc, slc)])
    pltpu.sync_copy(cnt_sh.at[pl.ds(sid * slc, slc)],
                    cnt_hbm.at[cid, pl.ds(sid * slc, slc)])


def _s2(users_p, scores, gmax_arr, ig):
    return pl.kernel(
        _s2_body,
        out_type=(
            jax.ShapeDtypeStruct((NC, U_PAD), jnp.float32),
            jax.ShapeDtypeStruct((NC, U_PAD), jnp.float32),
            jax.ShapeDtypeStruct((E_PAD, EMB), jnp.float32),
        ),
        mesh=_mesh(),
        scratch_types=[
            pltpu.VMEM((1, CHK), jnp.int32),
            pltpu.VMEM((CHK,), jnp.float32),
            pltpu.VMEM((L,), jnp.float32),
            pltpu.VMEM((1, CHK), jnp.float32),
            pltpu.VMEM((1, CHK), jnp.float32),
            pltpu.VMEM((U_PAD // NS,), jnp.float32),
            pltpu.VMEM((CHK, EMB), jnp.float32),
            pltpu.SemaphoreType.DMA,
            pltpu.VMEM_SHARED((U_PAD,), jnp.float32),
            pltpu.VMEM_SHARED((U_PAD,), jnp.float32),
        ],
    )(users_p, scores, gmax_arr, ig)


UHALF = U_PAD // NC
DUMP = 256
ACC_R = UHALF + DUMP


NR3 = (E_PAD // NS) // CHK


def _s3_body(users_hbm, wg_hbm, raw_hbm, ufl, bigidx, wbuf, zbuf, acc_sh, sem):
    cid = lax.axis_index("c")
    sid = lax.axis_index("s")
    base = sid * (E_PAD // NS)
    slc = UHALF // NS
    zslc = ACC_R // NS
    lo = cid * UHALF

    def zi(i, c):
        zbuf[i, pl.ds(0, L)] = jnp.zeros((L,), jnp.float32)
        return c
    lax.fori_loop(0, 100, zi, 0)

    def idxw(w, carry):
        off = base + w * W3
        pltpu.sync_copy(users_hbm.at[pl.ds(off, W3)], ufl)

        def vec(i, c3):
            u = ufl[pl.ds(i * L, L)]
            inh = (u >= lo) & (u < lo + UHALF)
            lu = jnp.where(inh, u - lo, UHALF + (u & (DUMP - 1)))
            bigidx[w * (W3 // CHK) + i // (CHK // L),
                   pl.ds((i % (CHK // L)) * L, L)] = lu
            return c3
        lax.fori_loop(0, W3 // L, vec, 0)
        return carry
    lax.fori_loop(0, (E_PAD // NS) // W3, idxw, 0)

    def rnd(rg, carry):
        def zc(i, c):
            pltpu.sync_copy(zbuf, acc_sh.at[pl.ds(sid * zslc + i * 100, 100)])
            return c
        lax.fori_loop(0, zslc // 100, zc, 0)
        pltpu.sync_copy(zbuf.at[pl.ds(0, 16)],
                        acc_sh.at[pl.ds(sid * zslc + (zslc // 100) * 100, 16)])
        plsc.subcore_barrier()
        c0 = rg * L

        def win(w, carry2):
            off = base + w * W3
            pltpu.sync_copy(wg_hbm.at[pl.ds(off, W3), pl.ds(c0, L)], wbuf)
            adds = []
            for q in range(W3 // CHK):
                adds.append(pltpu.async_copy(
                    wbuf.at[pl.ds(q * CHK, CHK)],
                    acc_sh.at[bigidx.at[w * (W3 // CHK) + q]],
                    sem, add=True))
            for a in adds:
                a.wait()
            return carry2
        lax.fori_loop(0, (E_PAD // NS) // W3, win, 0)

        plsc.subcore_barrier()
        pltpu.sync_copy(acc_sh.at[pl.ds(sid * slc, slc)],
                        raw_hbm.at[rg, cid, pl.ds(sid * slc, slc)])
        plsc.subcore_barrier()
        return carry

    lax.fori_loop(0, CG, rnd, 0)


def _s3(users_p, wg):
    return pl.kernel(
        _s3_body,
        out_type=jax.ShapeDtypeStruct((CG, NC, UHALF, L), jnp.float32),
        mesh=_mesh(),
        compiler_params=pltpu.CompilerParams(use_tc_tiling_on_sc=False),
        scratch_types=[
            pltpu.VMEM((W3,), jnp.int32),
            pltpu.VMEM((NR3, CHK), jnp.int32),
            pltpu.VMEM((W3, L), jnp.float32),
            pltpu.VMEM((100, L), jnp.float32),
            pltpu.VMEM_SHARED((ACC_R, L), jnp.float32),
            pltpu.SemaphoreType.DMA,
        ],
    )(users_p, wg)


def _t4_body(raw_ref, d_ref, c_ref, wv_ref, out_ref):
    r = raw_ref[...].reshape(CG, 1024, L)
    r = r.transpose(1, 0, 2).reshape(1024, EMB)
    agg = lax.dot_general(r, wv_ref[...], (((1,), (1,)), ((), ())),
                          preferred_element_type=jnp.float32)
    d = d_ref[0] + d_ref[1] + jnp.float32(1e-16)
    cnt = jnp.maximum(c_ref[0] + c_ref[1], 1.0)
    scale = 1.0 / (d * cnt)
    out_ref[...] = agg.reshape(8, 128, EMB) * scale[:, :, None]


def _t4(raw, den3, cnt3, Wv):
    n_blk = U_PAD // 1024
    return pl.pallas_call(
        _t4_body,
        grid=(n_blk,),
        in_specs=[
            pl.BlockSpec((CG, 1, 1024, L), lambda i: (0, i // 50, i % 50, 0)),
            pl.BlockSpec((NC, 8, 128), lambda i: (0, i, 0)),
            pl.BlockSpec((NC, 8, 128), lambda i: (0, i, 0)),
            pl.BlockSpec((EMB, EMB), lambda i: (0, 0)),
        ],
        out_specs=pl.BlockSpec((8, 128, EMB), lambda i: (i, 0, 0)),
        out_shape=jax.ShapeDtypeStruct((U_PAD // 128, 128, EMB), jnp.float32),
    )(raw, den3, cnt3, Wv)


@jax.jit
def kernel(inter_edge, user_embed, item_embed, Wq, Wk, Wv):
    users = inter_edge[0, :].astype(jnp.int32)
    items = inter_edge[1, :].astype(jnp.int32)
    pad = E_PAD - E
    pad_u = (jnp.arange(pad, dtype=jnp.int32) * 97) % NUM_USERS
    pad_i = (jnp.arange(pad, dtype=jnp.int32) * 89) % NUM_ITEMS
    users_p = jnp.concatenate([users, pad_u])
    items_p = jnp.concatenate([items, pad_i])

    ug, ig = _s0(users_p, items_p, user_embed, item_embed)
    scores3d, bmax = _t2(ug, ig, Wq, Wk)
    gmax = jnp.max(bmax)
    scores = scores3d.reshape(E_PAD)
    gmax_arr = jnp.full((L,), gmax, jnp.float32)
    den, cnt, wg = _s2(users_p, scores, gmax_arr, ig)
    raw = _s3(users_p, wg)
    den3 = den.reshape(NC, U_PAD // 128, 128)
    cnt3 = cnt.reshape(NC, U_PAD // 128, 128)
    out3 = _t4(raw, den3, cnt3, Wv)
    return out3.reshape(U_PAD, EMB)[:NUM_USERS]


# speedup vs baseline: 8.0089x; 1.1524x over previous
import functools

import jax
import jax.numpy as jnp
from jax import lax
from jax.experimental import pallas as pl
from jax.experimental.pallas import tpu as pltpu
from jax.experimental.pallas import tpu_sc as plsc

NUM_USERS = 100000
NUM_ITEMS = 100000
EMB = 128
E = 625000

NC = 2
NS = 16
NW = NC * NS
L = 16

B_PER_W = 19968
E_PAD = B_PER_W * NW
U_PAD = 102400
CHK = 128
W3 = 512
EBLK = 8192
NBLK = E_PAD // EBLK
CG = EMB // L

_mesh = functools.partial(plsc.VectorSubcoreMesh,
                          core_axis_name="c", subcore_axis_name="s")


def _wid():
    return lax.axis_index("s") * NC + lax.axis_index("c")


W0 = 256


def _s0_body(users_hbm, items_hbm, ue_hbm, ie_hbm, ug_hbm, ig_hbm,
             uidx, iidx, urows, irows, sem):
    wid = _wid()
    base = wid * B_PER_W

    def win(w, carry):
        off = base + w * W0
        cs = []
        for q in range(W0 // CHK):
            cs.append(pltpu.async_copy(
                users_hbm.at[pl.ds(off + q * CHK, CHK)], uidx.at[q], sem))
            cs.append(pltpu.async_copy(
                items_hbm.at[pl.ds(off + q * CHK, CHK)], iidx.at[q], sem))
        for c in cs:
            c.wait()
        gs = []
        for q in range(W0 // CHK):
            gs.append(pltpu.async_copy(
                ue_hbm.at[uidx.at[q]],
                urows.at[pl.ds(q * CHK, CHK)], sem))
            gs.append(pltpu.async_copy(
                ie_hbm.at[iidx.at[q]],
                irows.at[pl.ds(q * CHK, CHK)], sem))
        for g in gs:
            g.wait()
        w1 = pltpu.async_copy(urows, ug_hbm.at[pl.ds(off, W0)], sem)
        w2 = pltpu.async_copy(irows, ig_hbm.at[pl.ds(off, W0)], sem)
        w1.wait()
        w2.wait()
        return carry

    lax.fori_loop(0, B_PER_W // W0, win, 0)


def _s0(users_p, items_p, user_embed, item_embed):
    return pl.kernel(
        _s0_body,
        out_type=(
            jax.ShapeDtypeStruct((E_PAD, EMB), jnp.float32),
            jax.ShapeDtypeStruct((E_PAD, EMB), jnp.float32),
        ),
        mesh=_mesh(),
        scratch_types=[
            pltpu.VMEM((W0 // CHK, CHK), jnp.int32),
            pltpu.VMEM((W0 // CHK, CHK), jnp.int32),
            pltpu.VMEM((W0, EMB), jnp.float32),
            pltpu.VMEM((W0, EMB), jnp.float32),
            pltpu.SemaphoreType.DMA,
        ],
    )(users_p, items_p, user_embed, item_embed)


def _t2_body(ug_ref, ig_ref, wq_ref, wk_ref, s_ref, bm_ref):
    m = lax.dot_general(wq_ref[...], wk_ref[...], (((0,), (0,)), ((), ())),
                        preferred_element_type=jnp.float32)
    p = lax.dot_general(ug_ref[...], m, (((1,), (0,)), ((), ())),
                        preferred_element_type=jnp.float32)
    s = (p * ig_ref[...]).sum(axis=1) * jnp.float32(1.0 / (EMB ** 0.5))
    s_ref[...] = s[None, None, :]
    bm_ref[...] = jnp.full((1, 1, 128), jnp.max(s), jnp.float32)


def _t2(ug, ig, Wq, Wk):
    eb = pl.BlockSpec((EBLK, EMB), lambda i: (i, 0))
    w_spec = pl.BlockSpec((EMB, EMB), lambda i: (0, 0))
    return pl.pallas_call(
        _t2_body,
        grid=(NBLK,),
        in_specs=[eb, eb, w_spec, w_spec],
        out_specs=[pl.BlockSpec((1, 1, EBLK), lambda i: (i, 0, 0)),
                   pl.BlockSpec((1, 1, 128), lambda i: (i, 0, 0))],
        out_shape=[jax.ShapeDtypeStruct((NBLK, 1, EBLK), jnp.float32),
                   jax.ShapeDtypeStruct((NBLK, 1, 128), jnp.float32)],
    )(ug, ig, Wq, Wk)


def _s2_body(users_hbm, scores_hbm, gmax_hbm, ig_hbm,
             den_hbm, cnt_hbm, wg_hbm,
             ubuf, sbuf, gbuf, exbuf, onebuf, zbuf, igbuf, sem,
             den_sh, cnt_sh):
    wid = _wid()
    cid = lax.axis_index("c")
    sid = lax.axis_index("s")
    base = wid * B_PER_W
    slc = U_PAD // NS

    def z(i, c):
        zbuf[pl.ds(i * L, L)] = jnp.zeros((L,), jnp.float32)
        return c
    lax.fori_loop(0, slc // L, z, 0)
    pltpu.sync_copy(zbuf, den_sh.at[pl.ds(sid * slc, slc)])
    pltpu.sync_copy(zbuf, cnt_sh.at[pl.ds(sid * slc, slc)])
    pltpu.sync_copy(gmax_hbm, gbuf)
    plsc.subcore_barrier()
    g = gbuf[...]

    def win(w, carry):
        off = base + w * CHK
        c1 = pltpu.async_copy(users_hbm.at[pl.ds(off, CHK)], ubuf.at[0], sem)
        c2 = pltpu.async_copy(scores_hbm.at[pl.ds(off, CHK)], sbuf, sem)
        c3 = pltpu.async_copy(ig_hbm.at[pl.ds(off, CHK)], igbuf, sem)
        c1.wait()
        c2.wait()
        c3.wait()

        def vec(i, c2_):
            s = sbuf[pl.ds(i * L, L)]
            valid = (off + i * L + jax.lax.iota(jnp.int32, L)) < E
            ex = jnp.where(valid, jnp.exp(s - g), 0.0)
            one = jnp.where(valid, 1.0, 0.0).astype(jnp.float32)
            exbuf[0, pl.ds(i * L, L)] = ex
            onebuf[0, pl.ds(i * L, L)] = one
            for r16 in range(L):
                r = i * L + r16
                b = jnp.full((L,), ex[r16], jnp.float32)
                for j in range(EMB // L):
                    igbuf[r, pl.ds(j * L, L)] = igbuf[r, pl.ds(j * L, L)] * b
            return c2_
        lax.fori_loop(0, CHK // L, vec, 0)
        w1 = pltpu.async_copy(igbuf, wg_hbm.at[pl.ds(off, CHK)], sem)
        pltpu.sync_copy(exbuf.at[0], den_sh.at[ubuf.at[0]], add=True)
        pltpu.sync_copy(onebuf.at[0], cnt_sh.at[ubuf.at[0]], add=True)
        w1.wait()
        return carry

    lax.fori_loop(0, B_PER_W // CHK, win, 0)
    plsc.subcore_barrier()
    pltpu.sync_copy(den_sh.at[pl.ds(sid * slc, slc)],
                    den_hbm.at[cid, pl.ds(sid * slc, slc)])
    pltpu.sync_copy(cnt_sh.at[pl.ds(sid * slc, slc)],
                    cnt_hbm.at[cid, pl.ds(sid * slc, slc)])


def _s2(users_p, scores, gmax_arr, ig):
    return pl.kernel(
        _s2_body,
        out_type=(
            jax.ShapeDtypeStruct((NC, U_PAD), jnp.float32),
            jax.ShapeDtypeStruct((NC, U_PAD), jnp.float32),
            jax.ShapeDtypeStruct((E_PAD, EMB), jnp.float32),
        ),
        mesh=_mesh(),
        scratch_types=[
            pltpu.VMEM((1, CHK), jnp.int32),
            pltpu.VMEM((CHK,), jnp.float32),
            pltpu.VMEM((L,), jnp.float32),
            pltpu.VMEM((1, CHK), jnp.float32),
            pltpu.VMEM((1, CHK), jnp.float32),
            pltpu.VMEM((U_PAD // NS,), jnp.float32),
            pltpu.VMEM((CHK, EMB), jnp.float32),
            pltpu.SemaphoreType.DMA,
            pltpu.VMEM_SHARED((U_PAD,), jnp.float32),
            pltpu.VMEM_SHARED((U_PAD,), jnp.float32),
        ],
    )(users_p, scores, gmax_arr, ig)


UHALF = U_PAD // NC
DUMP = 256
ACC_R = UHALF + DUMP


NR3 = (E_PAD // NS) // CHK


def _s3_body(users_hbm, wg_hbm, raw_hbm, ufl, bigidx, wbuf, wbuf2, zbuf,
             acc_sh, sem, semA, semB):
    cid = lax.axis_index("c")
    sid = lax.axis_index("s")
    base = sid * (E_PAD // NS)
    slc = UHALF // NS
    zslc = ACC_R // NS
    lo = cid * UHALF

    def zi(i, c):
        zbuf[i, pl.ds(0, L)] = jnp.zeros((L,), jnp.float32)
        return c
    lax.fori_loop(0, 100, zi, 0)

    def idxw(w, carry):
        off = base + w * W3
        pltpu.sync_copy(users_hbm.at[pl.ds(off, W3)], ufl)

        def vec(i, c3):
            u = ufl[pl.ds(i * L, L)]
            inh = (u >= lo) & (u < lo + UHALF)
            lu = jnp.where(inh, u - lo, UHALF + (u & (DUMP - 1)))
            bigidx[w * (W3 // CHK) + i // (CHK // L),
                   pl.ds((i % (CHK // L)) * L, L)] = lu
            return c3
        lax.fori_loop(0, W3 // L, vec, 0)
        return carry
    lax.fori_loop(0, (E_PAD // NS) // W3, idxw, 0)

    def rnd(rg, carry):
        def zc(i, c):
            pltpu.sync_copy(zbuf, acc_sh.at[pl.ds(sid * zslc + i * 100, 100)])
            return c
        lax.fori_loop(0, zslc // 100, zc, 0)
        pltpu.sync_copy(zbuf.at[pl.ds(0, 16)],
                        acc_sh.at[pl.ds(sid * zslc + (zslc // 100) * 100, 16)])
        plsc.subcore_barrier()
        c0 = rg * L
        nwin = (E_PAD // NS) // W3

        def _src(w):
            return wg_hbm.at[pl.ds(base + w * W3, W3), pl.ds(c0, L)]

        def _adds(buf, w):
            adds = []
            for q in range(W3 // CHK):
                adds.append(pltpu.async_copy(
                    buf.at[pl.ds(q * CHK, CHK)],
                    acc_sh.at[bigidx.at[w * (W3 // CHK) + q]],
                    sem, add=True))
            for a in adds:
                a.wait()

        pltpu.async_copy(_src(0), wbuf, semA)

        def pair(p, carry2):
            w = 2 * p
            pltpu.make_async_copy(_src(w), wbuf, semA).wait()
            pltpu.async_copy(_src(w + 1), wbuf2, semB)
            _adds(wbuf, w)
            pltpu.make_async_copy(_src(w + 1), wbuf2, semB).wait()
            pltpu.async_copy(_src(lax.rem(w + 2, nwin)), wbuf, semA)
            _adds(wbuf2, w + 1)
            return carry2
        lax.fori_loop(0, nwin // 2, pair, 0)
        pltpu.make_async_copy(_src(0), wbuf, semA).wait()

        plsc.subcore_barrier()
        pltpu.sync_copy(acc_sh.at[pl.ds(sid * slc, slc)],
                        raw_hbm.at[rg, cid, pl.ds(sid * slc, slc)])
        plsc.subcore_barrier()
        return carry

    lax.fori_loop(0, CG, rnd, 0)


def _s3(users_p, wg):
    return pl.kernel(
        _s3_body,
        out_type=jax.ShapeDtypeStruct((CG, NC, UHALF, L), jnp.float32),
        mesh=_mesh(),
        compiler_params=pltpu.CompilerParams(use_tc_tiling_on_sc=False),
        scratch_types=[
            pltpu.VMEM((W3,), jnp.int32),
            pltpu.VMEM((NR3, CHK), jnp.int32),
            pltpu.VMEM((W3, L), jnp.float32),
            pltpu.VMEM((W3, L), jnp.float32),
            pltpu.VMEM((100, L), jnp.float32),
            pltpu.VMEM_SHARED((ACC_R, L), jnp.float32),
            pltpu.SemaphoreType.DMA,
            pltpu.SemaphoreType.DMA,
            pltpu.SemaphoreType.DMA,
        ],
    )(users_p, wg)


def _t4_body(raw_ref, d_ref, c_ref, wv_ref, out_ref):
    r = raw_ref[...].reshape(CG, 1024, L)
    r = r.transpose(1, 0, 2).reshape(1024, EMB)
    agg = lax.dot_general(r, wv_ref[...], (((1,), (1,)), ((), ())),
                          preferred_element_type=jnp.float32)
    d = d_ref[0] + d_ref[1] + jnp.float32(1e-16)
    cnt = jnp.maximum(c_ref[0] + c_ref[1], 1.0)
    scale = 1.0 / (d * cnt)
    out_ref[...] = agg.reshape(8, 128, EMB) * scale[:, :, None]


def _t4(raw, den3, cnt3, Wv):
    n_blk = U_PAD // 1024
    return pl.pallas_call(
        _t4_body,
        grid=(n_blk,),
        in_specs=[
            pl.BlockSpec((CG, 1, 1024, L), lambda i: (0, i // 50, i % 50, 0)),
            pl.BlockSpec((NC, 8, 128), lambda i: (0, i, 0)),
            pl.BlockSpec((NC, 8, 128), lambda i: (0, i, 0)),
            pl.BlockSpec((EMB, EMB), lambda i: (0, 0)),
        ],
        out_specs=pl.BlockSpec((8, 128, EMB), lambda i: (i, 0, 0)),
        out_shape=jax.ShapeDtypeStruct((U_PAD // 128, 128, EMB), jnp.float32),
    )(raw, den3, cnt3, Wv)


@jax.jit
def kernel(inter_edge, user_embed, item_embed, Wq, Wk, Wv):
    users = inter_edge[0, :].astype(jnp.int32)
    items = inter_edge[1, :].astype(jnp.int32)
    pad = E_PAD - E
    pad_u = (jnp.arange(pad, dtype=jnp.int32) * 97) % NUM_USERS
    pad_i = (jnp.arange(pad, dtype=jnp.int32) * 89) % NUM_ITEMS
    users_p = jnp.concatenate([users, pad_u])
    items_p = jnp.concatenate([items, pad_i])

    ug, ig = _s0(users_p, items_p, user_embed, item_embed)
    scores3d, bmax = _t2(ug, ig, Wq, Wk)
    gmax = jnp.max(bmax)
    scores = scores3d.reshape(E_PAD)
    gmax_arr = jnp.full((L,), gmax, jnp.float32)
    den, cnt, wg = _s2(users_p, scores, gmax_arr, ig)
    raw = _s3(users_p, wg)
    den3 = den.reshape(NC, U_PAD // 128, 128)
    cnt3 = cnt.reshape(NC, U_PAD // 128, 128)
    out3 = _t4(raw, den3, cnt3, Wv)
    return out3.reshape(U_PAD, EMB)[:NUM_USERS]


# speedup vs baseline: 8.5713x; 1.0702x over previous
import functools

import jax
import jax.numpy as jnp
from jax import lax
from jax.experimental import pallas as pl
from jax.experimental.pallas import tpu as pltpu
from jax.experimental.pallas import tpu_sc as plsc

NUM_USERS = 100000
NUM_ITEMS = 100000
EMB = 128
E = 625000

NC = 2
NS = 16
NW = NC * NS
L = 16

B_PER_W = 19968
E_PAD = B_PER_W * NW
U_PAD = 102400
CHK = 128
W3 = 512
EBLK = 8192
NBLK = E_PAD // EBLK
CG = EMB // L

_mesh = functools.partial(plsc.VectorSubcoreMesh,
                          core_axis_name="c", subcore_axis_name="s")


def _wid():
    return lax.axis_index("s") * NC + lax.axis_index("c")


W0 = 256


def _s0_body(users_hbm, items_hbm, ue_hbm, ie_hbm, ug_hbm, ig_hbm,
             uidx, iidx, urows, irows, sem):
    wid = _wid()
    base = wid * B_PER_W

    def win(w, carry):
        off = base + w * W0
        cs = []
        for q in range(W0 // CHK):
            cs.append(pltpu.async_copy(
                users_hbm.at[pl.ds(off + q * CHK, CHK)], uidx.at[q], sem))
            cs.append(pltpu.async_copy(
                items_hbm.at[pl.ds(off + q * CHK, CHK)], iidx.at[q], sem))
        for c in cs:
            c.wait()
        gs = []
        for q in range(W0 // CHK):
            gs.append(pltpu.async_copy(
                ue_hbm.at[uidx.at[q]],
                urows.at[pl.ds(q * CHK, CHK)], sem))
            gs.append(pltpu.async_copy(
                ie_hbm.at[iidx.at[q]],
                irows.at[pl.ds(q * CHK, CHK)], sem))
        for g in gs:
            g.wait()
        w1 = pltpu.async_copy(urows, ug_hbm.at[pl.ds(off, W0)], sem)
        w2 = pltpu.async_copy(irows, ig_hbm.at[pl.ds(off, W0)], sem)
        w1.wait()
        w2.wait()
        return carry

    lax.fori_loop(0, B_PER_W // W0, win, 0)


def _s0(users_p, items_p, user_embed, item_embed):
    return pl.kernel(
        _s0_body,
        out_type=(
            jax.ShapeDtypeStruct((E_PAD, EMB), jnp.float32),
            jax.ShapeDtypeStruct((E_PAD, EMB), jnp.float32),
        ),
        mesh=_mesh(),
        scratch_types=[
            pltpu.VMEM((W0 // CHK, CHK), jnp.int32),
            pltpu.VMEM((W0 // CHK, CHK), jnp.int32),
            pltpu.VMEM((W0, EMB), jnp.float32),
            pltpu.VMEM((W0, EMB), jnp.float32),
            pltpu.SemaphoreType.DMA,
        ],
    )(users_p, items_p, user_embed, item_embed)


def _t2_body(ug_ref, ig_ref, wq_ref, wk_ref, s_ref, bm_ref):
    m = lax.dot_general(wq_ref[...], wk_ref[...], (((0,), (0,)), ((), ())),
                        preferred_element_type=jnp.float32)
    p = lax.dot_general(ug_ref[...], m, (((1,), (0,)), ((), ())),
                        preferred_element_type=jnp.float32)
    s = (p * ig_ref[...]).sum(axis=1) * jnp.float32(1.0 / (EMB ** 0.5))
    s_ref[...] = s[None, None, :]
    bm_ref[...] = jnp.full((1, 1, 128), jnp.max(s), jnp.float32)


def _t2(ug, ig, Wq, Wk):
    eb = pl.BlockSpec((EBLK, EMB), lambda i: (i, 0))
    w_spec = pl.BlockSpec((EMB, EMB), lambda i: (0, 0))
    return pl.pallas_call(
        _t2_body,
        grid=(NBLK,),
        in_specs=[eb, eb, w_spec, w_spec],
        out_specs=[pl.BlockSpec((1, 1, EBLK), lambda i: (i, 0, 0)),
                   pl.BlockSpec((1, 1, 128), lambda i: (i, 0, 0))],
        out_shape=[jax.ShapeDtypeStruct((NBLK, 1, EBLK), jnp.float32),
                   jax.ShapeDtypeStruct((NBLK, 1, 128), jnp.float32)],
    )(ug, ig, Wq, Wk)


def _s2_body(users_hbm, scores_hbm, gmax_hbm, ig_hbm,
             den_hbm, cnt_hbm, wg_hbm,
             ubuf, sbuf, gbuf, exbuf, onebuf, zbuf, igbuf,
             ubuf2, sbuf2, igbuf2, semA, semB, semW,
             den_sh, cnt_sh):
    wid = _wid()
    cid = lax.axis_index("c")
    sid = lax.axis_index("s")
    base = wid * B_PER_W
    slc = U_PAD // NS

    def z(i, c):
        zbuf[pl.ds(i * L, L)] = jnp.zeros((L,), jnp.float32)
        return c
    lax.fori_loop(0, slc // L, z, 0)
    pltpu.sync_copy(zbuf, den_sh.at[pl.ds(sid * slc, slc)])
    pltpu.sync_copy(zbuf, cnt_sh.at[pl.ds(sid * slc, slc)])
    pltpu.sync_copy(gmax_hbm, gbuf)
    plsc.subcore_barrier()
    g = gbuf[...]

    nwin = B_PER_W // CHK

    def _starts(w, ub, sb, ib, sm):
        off = base + lax.rem(w, nwin) * CHK
        pltpu.async_copy(users_hbm.at[pl.ds(off, CHK)], ub.at[0], sm)
        pltpu.async_copy(scores_hbm.at[pl.ds(off, CHK)], sb, sm)
        pltpu.async_copy(ig_hbm.at[pl.ds(off, CHK)], ib, sm)

    def _waits(w, ub, sb, ib, sm):
        off = base + lax.rem(w, nwin) * CHK
        pltpu.make_async_copy(users_hbm.at[pl.ds(off, CHK)], ub.at[0], sm).wait()
        pltpu.make_async_copy(scores_hbm.at[pl.ds(off, CHK)], sb, sm).wait()
        pltpu.make_async_copy(ig_hbm.at[pl.ds(off, CHK)], ib, sm).wait()

    def _process(w, ub, sb, ib):
        off = base + w * CHK

        def vec(i, c2_):
            s = sb[pl.ds(i * L, L)]
            valid = (off + i * L + jax.lax.iota(jnp.int32, L)) < E
            ex = jnp.where(valid, jnp.exp(s - g), 0.0)
            one = jnp.where(valid, 1.0, 0.0).astype(jnp.float32)
            exbuf[0, pl.ds(i * L, L)] = ex
            onebuf[0, pl.ds(i * L, L)] = one
            for r16 in range(L):
                r = i * L + r16
                b = jnp.full((L,), ex[r16], jnp.float32)
                for j in range(EMB // L):
                    ib[r, pl.ds(j * L, L)] = ib[r, pl.ds(j * L, L)] * b
            return c2_
        lax.fori_loop(0, CHK // L, vec, 0)
        wr = pltpu.async_copy(ib, wg_hbm.at[pl.ds(off, CHK)], semW)
        pltpu.sync_copy(exbuf.at[0], den_sh.at[ub.at[0]], add=True)
        pltpu.sync_copy(onebuf.at[0], cnt_sh.at[ub.at[0]], add=True)
        wr.wait()

    _starts(0, ubuf, sbuf, igbuf, semA)

    def pair(p, carry):
        w = 2 * p
        _waits(w, ubuf, sbuf, igbuf, semA)
        _starts(w + 1, ubuf2, sbuf2, igbuf2, semB)
        _process(w, ubuf, sbuf, igbuf)
        _waits(w + 1, ubuf2, sbuf2, igbuf2, semB)
        _starts(w + 2, ubuf, sbuf, igbuf, semA)
        _process(w + 1, ubuf2, sbuf2, igbuf2)
        return carry
    lax.fori_loop(0, nwin // 2, pair, 0)
    _waits(0, ubuf, sbuf, igbuf, semA)
    plsc.subcore_barrier()
    pltpu.sync_copy(den_sh.at[pl.ds(sid * slc, slc)],
                    den_hbm.at[cid, pl.ds(sid * slc, slc)])
    pltpu.sync_copy(cnt_sh.at[pl.ds(sid * slc, slc)],
                    cnt_hbm.at[cid, pl.ds(sid * slc, slc)])


def _s2(users_p, scores, gmax_arr, ig):
    return pl.kernel(
        _s2_body,
        out_type=(
            jax.ShapeDtypeStruct((NC, U_PAD), jnp.float32),
            jax.ShapeDtypeStruct((NC, U_PAD), jnp.float32),
            jax.ShapeDtypeStruct((E_PAD, EMB), jnp.float32),
        ),
        mesh=_mesh(),
        scratch_types=[
            pltpu.VMEM((1, CHK), jnp.int32),
            pltpu.VMEM((CHK,), jnp.float32),
            pltpu.VMEM((L,), jnp.float32),
            pltpu.VMEM((1, CHK), jnp.float32),
            pltpu.VMEM((1, CHK), jnp.float32),
            pltpu.VMEM((U_PAD // NS,), jnp.float32),
            pltpu.VMEM((CHK, EMB), jnp.float32),
            pltpu.VMEM((1, CHK), jnp.int32),
            pltpu.VMEM((CHK,), jnp.float32),
            pltpu.VMEM((CHK, EMB), jnp.float32),
            pltpu.SemaphoreType.DMA,
            pltpu.SemaphoreType.DMA,
            pltpu.SemaphoreType.DMA,
            pltpu.VMEM_SHARED((U_PAD,), jnp.float32),
            pltpu.VMEM_SHARED((U_PAD,), jnp.float32),
        ],
    )(users_p, scores, gmax_arr, ig)


UHALF = U_PAD // NC
DUMP = 256
ACC_R = UHALF + DUMP


NR3 = (E_PAD // NS) // CHK


def _s3_body(users_hbm, wg_hbm, raw_hbm, ufl, bigidx, wbuf, wbuf2, zbuf,
             acc_sh, sem, semA, semB):
    cid = lax.axis_index("c")
    sid = lax.axis_index("s")
    base = sid * (E_PAD // NS)
    slc = UHALF // NS
    zslc = ACC_R // NS
    lo = cid * UHALF

    def zi(i, c):
        zbuf[i, pl.ds(0, L)] = jnp.zeros((L,), jnp.float32)
        return c
    lax.fori_loop(0, 100, zi, 0)

    def idxw(w, carry):
        off = base + w * W3
        pltpu.sync_copy(users_hbm.at[pl.ds(off, W3)], ufl)

        def vec(i, c3):
            u = ufl[pl.ds(i * L, L)]
            inh = (u >= lo) & (u < lo + UHALF)
            lu = jnp.where(inh, u - lo, UHALF + (u & (DUMP - 1)))
            bigidx[w * (W3 // CHK) + i // (CHK // L),
                   pl.ds((i % (CHK // L)) * L, L)] = lu
            return c3
        lax.fori_loop(0, W3 // L, vec, 0)
        return carry
    lax.fori_loop(0, (E_PAD // NS) // W3, idxw, 0)

    def rnd(rg, carry):
        def zc(i, c):
            pltpu.sync_copy(zbuf, acc_sh.at[pl.ds(sid * zslc + i * 100, 100)])
            return c
        lax.fori_loop(0, zslc // 100, zc, 0)
        pltpu.sync_copy(zbuf.at[pl.ds(0, 16)],
                        acc_sh.at[pl.ds(sid * zslc + (zslc // 100) * 100, 16)])
        plsc.subcore_barrier()
        c0 = rg * L
        nwin = (E_PAD // NS) // W3

        def _src(w):
            return wg_hbm.at[pl.ds(base + w * W3, W3), pl.ds(c0, L)]

        def _adds(buf, w):
            adds = []
            for q in range(W3 // CHK):
                adds.append(pltpu.async_copy(
                    buf.at[pl.ds(q * CHK, CHK)],
                    acc_sh.at[bigidx.at[w * (W3 // CHK) + q]],
                    sem, add=True))
            for a in adds:
                a.wait()

        pltpu.async_copy(_src(0), wbuf, semA)

        def pair(p, carry2):
            w = 2 * p
            pltpu.make_async_copy(_src(w), wbuf, semA).wait()
            pltpu.async_copy(_src(w + 1), wbuf2, semB)
            _adds(wbuf, w)
            pltpu.make_async_copy(_src(w + 1), wbuf2, semB).wait()
            pltpu.async_copy(_src(lax.rem(w + 2, nwin)), wbuf, semA)
            _adds(wbuf2, w + 1)
            return carry2
        lax.fori_loop(0, nwin // 2, pair, 0)
        pltpu.make_async_copy(_src(0), wbuf, semA).wait()

        plsc.subcore_barrier()
        pltpu.sync_copy(acc_sh.at[pl.ds(sid * slc, slc)],
                        raw_hbm.at[rg, cid, pl.ds(sid * slc, slc)])
        plsc.subcore_barrier()
        return carry

    lax.fori_loop(0, CG, rnd, 0)


def _s3(users_p, wg):
    return pl.kernel(
        _s3_body,
        out_type=jax.ShapeDtypeStruct((CG, NC, UHALF, L), jnp.float32),
        mesh=_mesh(),
        compiler_params=pltpu.CompilerParams(use_tc_tiling_on_sc=False),
        scratch_types=[
            pltpu.VMEM((W3,), jnp.int32),
            pltpu.VMEM((NR3, CHK), jnp.int32),
            pltpu.VMEM((W3, L), jnp.float32),
            pltpu.VMEM((W3, L), jnp.float32),
            pltpu.VMEM((100, L), jnp.float32),
            pltpu.VMEM_SHARED((ACC_R, L), jnp.float32),
            pltpu.SemaphoreType.DMA,
            pltpu.SemaphoreType.DMA,
            pltpu.SemaphoreType.DMA,
        ],
    )(users_p, wg)


def _t4_body(raw_ref, d_ref, c_ref, wv_ref, out_ref):
    r = raw_ref[...].reshape(CG, 1024, L)
    r = r.transpose(1, 0, 2).reshape(1024, EMB)
    agg = lax.dot_general(r, wv_ref[...], (((1,), (1,)), ((), ())),
                          preferred_element_type=jnp.float32)
    d = d_ref[0] + d_ref[1] + jnp.float32(1e-16)
    cnt = jnp.maximum(c_ref[0] + c_ref[1], 1.0)
    scale = 1.0 / (d * cnt)
    out_ref[...] = agg.reshape(8, 128, EMB) * scale[:, :, None]


def _t4(raw, den3, cnt3, Wv):
    n_blk = U_PAD // 1024
    return pl.pallas_call(
        _t4_body,
        grid=(n_blk,),
        in_specs=[
            pl.BlockSpec((CG, 1, 1024, L), lambda i: (0, i // 50, i % 50, 0)),
            pl.BlockSpec((NC, 8, 128), lambda i: (0, i, 0)),
            pl.BlockSpec((NC, 8, 128), lambda i: (0, i, 0)),
            pl.BlockSpec((EMB, EMB), lambda i: (0, 0)),
        ],
        out_specs=pl.BlockSpec((8, 128, EMB), lambda i: (i, 0, 0)),
        out_shape=jax.ShapeDtypeStruct((U_PAD // 128, 128, EMB), jnp.float32),
    )(raw, den3, cnt3, Wv)


@jax.jit
def kernel(inter_edge, user_embed, item_embed, Wq, Wk, Wv):
    users = inter_edge[0, :].astype(jnp.int32)
    items = inter_edge[1, :].astype(jnp.int32)
    pad = E_PAD - E
    pad_u = (jnp.arange(pad, dtype=jnp.int32) * 97) % NUM_USERS
    pad_i = (jnp.arange(pad, dtype=jnp.int32) * 89) % NUM_ITEMS
    users_p = jnp.concatenate([users, pad_u])
    items_p = jnp.concatenate([items, pad_i])

    ug, ig = _s0(users_p, items_p, user_embed, item_embed)
    scores3d, bmax = _t2(ug, ig, Wq, Wk)
    gmax = jnp.max(bmax)
    scores = scores3d.reshape(E_PAD)
    gmax_arr = jnp.full((L,), gmax, jnp.float32)
    den, cnt, wg = _s2(users_p, scores, gmax_arr, ig)
    raw = _s3(users_p, wg)
    den3 = den.reshape(NC, U_PAD // 128, 128)
    cnt3 = cnt.reshape(NC, U_PAD // 128, 128)
    out3 = _t4(raw, den3, cnt3, Wv)
    return out3.reshape(U_PAD, EMB)[:NUM_USERS]


# speedup vs baseline: 8.6163x; 1.0052x over previous
import functools

import jax
import jax.numpy as jnp
from jax import lax
from jax.experimental import pallas as pl
from jax.experimental.pallas import tpu as pltpu
from jax.experimental.pallas import tpu_sc as plsc

NUM_USERS = 100000
NUM_ITEMS = 100000
EMB = 128
E = 625000

NC = 2
NS = 16
NW = NC * NS
L = 16

B_PER_W = 19968
E_PAD = B_PER_W * NW
U_PAD = 102400
CHK = 128
W3 = 512
EBLK = 8192
NBLK = E_PAD // EBLK
CG = EMB // L

_mesh = functools.partial(plsc.VectorSubcoreMesh,
                          core_axis_name="c", subcore_axis_name="s")


def _wid():
    return lax.axis_index("s") * NC + lax.axis_index("c")


W0 = 128


def _s0_body(users_hbm, items_hbm, ue_hbm, ie_hbm, ug_hbm, ig_hbm,
             uidx, iidx, urows0, irows0, urows1, irows1, semI, semG, semW):
    wid = _wid()
    base = wid * B_PER_W
    nwin = B_PER_W // W0

    def _start_idx(w):
        off = base + lax.rem(w, nwin) * W0
        pltpu.async_copy(users_hbm.at[pl.ds(off, W0)], uidx.at[w % 2], semI)
        pltpu.async_copy(items_hbm.at[pl.ds(off, W0)], iidx.at[w % 2], semI)

    def _wait_idx(w):
        off = base + lax.rem(w, nwin) * W0
        pltpu.make_async_copy(users_hbm.at[pl.ds(off, W0)],
                              uidx.at[w % 2], semI).wait()
        pltpu.make_async_copy(items_hbm.at[pl.ds(off, W0)],
                              iidx.at[w % 2], semI).wait()

    def _gather(w, ur, ir):
        g1 = pltpu.async_copy(ue_hbm.at[uidx.at[w % 2]], ur, semG)
        g2 = pltpu.async_copy(ie_hbm.at[iidx.at[w % 2]], ir, semG)
        g1.wait()
        g2.wait()

    def _start_write(w, ur, ir):
        off = base + w * W0
        pltpu.async_copy(ur, ug_hbm.at[pl.ds(off, W0)], semW)
        pltpu.async_copy(ir, ig_hbm.at[pl.ds(off, W0)], semW)

    def _wait_write(w, ur, ir):
        off = base + w * W0
        pltpu.make_async_copy(ur, ug_hbm.at[pl.ds(off, W0)], semW).wait()
        pltpu.make_async_copy(ir, ig_hbm.at[pl.ds(off, W0)], semW).wait()

    _start_idx(0)

    def pair(p, carry):
        w = 2 * p
        _wait_idx(w)
        _start_idx(w + 1)
        _gather(w, urows0, irows0)
        _start_write(w, urows0, irows0)
        _wait_idx(w + 1)
        _start_idx(w + 2)
        _gather(w + 1, urows1, irows1)
        _start_write(w + 1, urows1, irows1)
        _wait_write(w, urows0, irows0)
        _wait_write(w + 1, urows1, irows1)
        return carry

    lax.fori_loop(0, nwin // 2, pair, 0)
    _wait_idx(0)


def _s0(users_p, items_p, user_embed, item_embed):
    return pl.kernel(
        _s0_body,
        out_type=(
            jax.ShapeDtypeStruct((E_PAD, EMB), jnp.float32),
            jax.ShapeDtypeStruct((E_PAD, EMB), jnp.float32),
        ),
        mesh=_mesh(),
        scratch_types=[
            pltpu.VMEM((2, CHK), jnp.int32),
            pltpu.VMEM((2, CHK), jnp.int32),
            pltpu.VMEM((W0, EMB), jnp.float32),
            pltpu.VMEM((W0, EMB), jnp.float32),
            pltpu.VMEM((W0, EMB), jnp.float32),
            pltpu.VMEM((W0, EMB), jnp.float32),
            pltpu.SemaphoreType.DMA,
            pltpu.SemaphoreType.DMA,
            pltpu.SemaphoreType.DMA,
        ],
    )(users_p, items_p, user_embed, item_embed)


def _t2_body(ug_ref, ig_ref, wq_ref, wk_ref, s_ref, bm_ref):
    m = lax.dot_general(wq_ref[...], wk_ref[...], (((0,), (0,)), ((), ())),
                        preferred_element_type=jnp.float32)
    p = lax.dot_general(ug_ref[...], m, (((1,), (0,)), ((), ())),
                        preferred_element_type=jnp.float32)
    s = (p * ig_ref[...]).sum(axis=1) * jnp.float32(1.0 / (EMB ** 0.5))
    s_ref[...] = s[None, None, :]
    bm_ref[...] = jnp.full((1, 1, 128), jnp.max(s), jnp.float32)


def _t2(ug, ig, Wq, Wk):
    eb = pl.BlockSpec((EBLK, EMB), lambda i: (i, 0))
    w_spec = pl.BlockSpec((EMB, EMB), lambda i: (0, 0))
    return pl.pallas_call(
        _t2_body,
        grid=(NBLK,),
        in_specs=[eb, eb, w_spec, w_spec],
        out_specs=[pl.BlockSpec((1, 1, EBLK), lambda i: (i, 0, 0)),
                   pl.BlockSpec((1, 1, 128), lambda i: (i, 0, 0))],
        out_shape=[jax.ShapeDtypeStruct((NBLK, 1, EBLK), jnp.float32),
                   jax.ShapeDtypeStruct((NBLK, 1, 128), jnp.float32)],
    )(ug, ig, Wq, Wk)


def _s2_body(users_hbm, scores_hbm, gmax_hbm, ig_hbm,
             den_hbm, cnt_hbm, wg_hbm,
             ubuf, sbuf, gbuf, exbuf, onebuf, zbuf, igbuf,
             ubuf2, sbuf2, igbuf2, semA, semB, semW,
             den_sh, cnt_sh):
    wid = _wid()
    cid = lax.axis_index("c")
    sid = lax.axis_index("s")
    base = wid * B_PER_W
    slc = U_PAD // NS

    def z(i, c):
        zbuf[pl.ds(i * L, L)] = jnp.zeros((L,), jnp.float32)
        return c
    lax.fori_loop(0, slc // L, z, 0)
    pltpu.sync_copy(zbuf, den_sh.at[pl.ds(sid * slc, slc)])
    pltpu.sync_copy(zbuf, cnt_sh.at[pl.ds(sid * slc, slc)])
    pltpu.sync_copy(gmax_hbm, gbuf)
    plsc.subcore_barrier()
    g = gbuf[...]

    nwin = B_PER_W // CHK

    def _starts(w, ub, sb, ib, sm):
        off = base + lax.rem(w, nwin) * CHK
        pltpu.async_copy(users_hbm.at[pl.ds(off, CHK)], ub.at[0], sm)
        pltpu.async_copy(scores_hbm.at[pl.ds(off, CHK)], sb, sm)
        pltpu.async_copy(ig_hbm.at[pl.ds(off, CHK)], ib, sm)

    def _waits(w, ub, sb, ib, sm):
        off = base + lax.rem(w, nwin) * CHK
        pltpu.make_async_copy(users_hbm.at[pl.ds(off, CHK)], ub.at[0], sm).wait()
        pltpu.make_async_copy(scores_hbm.at[pl.ds(off, CHK)], sb, sm).wait()
        pltpu.make_async_copy(ig_hbm.at[pl.ds(off, CHK)], ib, sm).wait()

    def _process(w, ub, sb, ib):
        off = base + w * CHK

        def vec(i, c2_):
            s = sb[pl.ds(i * L, L)]
            valid = (off + i * L + jax.lax.iota(jnp.int32, L)) < E
            ex = jnp.where(valid, jnp.exp(s - g), 0.0)
            one = jnp.where(valid, 1.0, 0.0).astype(jnp.float32)
            exbuf[0, pl.ds(i * L, L)] = ex
            onebuf[0, pl.ds(i * L, L)] = one
            for r16 in range(L):
                r = i * L + r16
                b = jnp.full((L,), ex[r16], jnp.float32)
                for j in range(EMB // L):
                    ib[r, pl.ds(j * L, L)] = ib[r, pl.ds(j * L, L)] * b
            return c2_
        lax.fori_loop(0, CHK // L, vec, 0)
        wr = pltpu.async_copy(ib, wg_hbm.at[pl.ds(off, CHK)], semW)
        pltpu.sync_copy(exbuf.at[0], den_sh.at[ub.at[0]], add=True)
        pltpu.sync_copy(onebuf.at[0], cnt_sh.at[ub.at[0]], add=True)
        wr.wait()

    _starts(0, ubuf, sbuf, igbuf, semA)

    def pair(p, carry):
        w = 2 * p
        _waits(w, ubuf, sbuf, igbuf, semA)
        _starts(w + 1, ubuf2, sbuf2, igbuf2, semB)
        _process(w, ubuf, sbuf, igbuf)
        _waits(w + 1, ubuf2, sbuf2, igbuf2, semB)
        _starts(w + 2, ubuf, sbuf, igbuf, semA)
        _process(w + 1, ubuf2, sbuf2, igbuf2)
        return carry
    lax.fori_loop(0, nwin // 2, pair, 0)
    _waits(0, ubuf, sbuf, igbuf, semA)
    plsc.subcore_barrier()
    pltpu.sync_copy(den_sh.at[pl.ds(sid * slc, slc)],
                    den_hbm.at[cid, pl.ds(sid * slc, slc)])
    pltpu.sync_copy(cnt_sh.at[pl.ds(sid * slc, slc)],
                    cnt_hbm.at[cid, pl.ds(sid * slc, slc)])


def _s2(users_p, scores, gmax_arr, ig):
    return pl.kernel(
        _s2_body,
        out_type=(
            jax.ShapeDtypeStruct((NC, U_PAD), jnp.float32),
            jax.ShapeDtypeStruct((NC, U_PAD), jnp.float32),
            jax.ShapeDtypeStruct((E_PAD, EMB), jnp.float32),
        ),
        mesh=_mesh(),
        scratch_types=[
            pltpu.VMEM((1, CHK), jnp.int32),
            pltpu.VMEM((CHK,), jnp.float32),
            pltpu.VMEM((L,), jnp.float32),
            pltpu.VMEM((1, CHK), jnp.float32),
            pltpu.VMEM((1, CHK), jnp.float32),
            pltpu.VMEM((U_PAD // NS,), jnp.float32),
            pltpu.VMEM((CHK, EMB), jnp.float32),
            pltpu.VMEM((1, CHK), jnp.int32),
            pltpu.VMEM((CHK,), jnp.float32),
            pltpu.VMEM((CHK, EMB), jnp.float32),
            pltpu.SemaphoreType.DMA,
            pltpu.SemaphoreType.DMA,
            pltpu.SemaphoreType.DMA,
            pltpu.VMEM_SHARED((U_PAD,), jnp.float32),
            pltpu.VMEM_SHARED((U_PAD,), jnp.float32),
        ],
    )(users_p, scores, gmax_arr, ig)


UHALF = U_PAD // NC
DUMP = 256
ACC_R = UHALF + DUMP


NR3 = (E_PAD // NS) // CHK


def _s3_body(users_hbm, wg_hbm, raw_hbm, ufl, bigidx, wbuf, wbuf2, zbuf,
             acc_sh, sem, semA, semB):
    cid = lax.axis_index("c")
    sid = lax.axis_index("s")
    base = sid * (E_PAD // NS)
    slc = UHALF // NS
    zslc = ACC_R // NS
    lo = cid * UHALF

    def zi(i, c):
        zbuf[i, pl.ds(0, L)] = jnp.zeros((L,), jnp.float32)
        return c
    lax.fori_loop(0, 100, zi, 0)

    def idxw(w, carry):
        off = base + w * W3
        pltpu.sync_copy(users_hbm.at[pl.ds(off, W3)], ufl)

        def vec(i, c3):
            u = ufl[pl.ds(i * L, L)]
            inh = (u >= lo) & (u < lo + UHALF)
            lu = jnp.where(inh, u - lo, UHALF + (u & (DUMP - 1)))
            bigidx[w * (W3 // CHK) + i // (CHK // L),
                   pl.ds((i % (CHK // L)) * L, L)] = lu
            return c3
        lax.fori_loop(0, W3 // L, vec, 0)
        return carry
    lax.fori_loop(0, (E_PAD // NS) // W3, idxw, 0)

    def rnd(rg, carry):
        def zc(i, c):
            pltpu.sync_copy(zbuf, acc_sh.at[pl.ds(sid * zslc + i * 100, 100)])
            return c
        lax.fori_loop(0, zslc // 100, zc, 0)
        pltpu.sync_copy(zbuf.at[pl.ds(0, 16)],
                        acc_sh.at[pl.ds(sid * zslc + (zslc // 100) * 100, 16)])
        plsc.subcore_barrier()
        c0 = rg * L
        nwin = (E_PAD // NS) // W3

        def _src(w):
            return wg_hbm.at[pl.ds(base + w * W3, W3), pl.ds(c0, L)]

        def _adds(buf, w):
            adds = []
            for q in range(W3 // CHK):
                adds.append(pltpu.async_copy(
                    buf.at[pl.ds(q * CHK, CHK)],
                    acc_sh.at[bigidx.at[w * (W3 // CHK) + q]],
                    sem, add=True))
            for a in adds:
                a.wait()

        pltpu.async_copy(_src(0), wbuf, semA)

        def pair(p, carry2):
            w = 2 * p
            pltpu.make_async_copy(_src(w), wbuf, semA).wait()
            pltpu.async_copy(_src(w + 1), wbuf2, semB)
            _adds(wbuf, w)
            pltpu.make_async_copy(_src(w + 1), wbuf2, semB).wait()
            pltpu.async_copy(_src(lax.rem(w + 2, nwin)), wbuf, semA)
            _adds(wbuf2, w + 1)
            return carry2
        lax.fori_loop(0, nwin // 2, pair, 0)
        pltpu.make_async_copy(_src(0), wbuf, semA).wait()

        plsc.subcore_barrier()
        pltpu.sync_copy(acc_sh.at[pl.ds(sid * slc, slc)],
                        raw_hbm.at[rg, cid, pl.ds(sid * slc, slc)])
        plsc.subcore_barrier()
        return carry

    lax.fori_loop(0, CG, rnd, 0)


def _s3(users_p, wg):
    return pl.kernel(
        _s3_body,
        out_type=jax.ShapeDtypeStruct((CG, NC, UHALF, L), jnp.float32),
        mesh=_mesh(),
        compiler_params=pltpu.CompilerParams(use_tc_tiling_on_sc=False),
        scratch_types=[
            pltpu.VMEM((W3,), jnp.int32),
            pltpu.VMEM((NR3, CHK), jnp.int32),
            pltpu.VMEM((W3, L), jnp.float32),
            pltpu.VMEM((W3, L), jnp.float32),
            pltpu.VMEM((100, L), jnp.float32),
            pltpu.VMEM_SHARED((ACC_R, L), jnp.float32),
            pltpu.SemaphoreType.DMA,
            pltpu.SemaphoreType.DMA,
            pltpu.SemaphoreType.DMA,
        ],
    )(users_p, wg)


def _t4_body(raw_ref, d_ref, c_ref, wv_ref, out_ref):
    r = raw_ref[...].reshape(CG, 1024, L)
    r = r.transpose(1, 0, 2).reshape(1024, EMB)
    agg = lax.dot_general(r, wv_ref[...], (((1,), (1,)), ((), ())),
                          preferred_element_type=jnp.float32)
    d = d_ref[0] + d_ref[1] + jnp.float32(1e-16)
    cnt = jnp.maximum(c_ref[0] + c_ref[1], 1.0)
    scale = 1.0 / (d * cnt)
    out_ref[...] = agg.reshape(8, 128, EMB) * scale[:, :, None]


def _t4(raw, den3, cnt3, Wv):
    n_blk = U_PAD // 1024
    return pl.pallas_call(
        _t4_body,
        grid=(n_blk,),
        in_specs=[
            pl.BlockSpec((CG, 1, 1024, L), lambda i: (0, i // 50, i % 50, 0)),
            pl.BlockSpec((NC, 8, 128), lambda i: (0, i, 0)),
            pl.BlockSpec((NC, 8, 128), lambda i: (0, i, 0)),
            pl.BlockSpec((EMB, EMB), lambda i: (0, 0)),
        ],
        out_specs=pl.BlockSpec((8, 128, EMB), lambda i: (i, 0, 0)),
        out_shape=jax.ShapeDtypeStruct((U_PAD // 128, 128, EMB), jnp.float32),
    )(raw, den3, cnt3, Wv)


@jax.jit
def kernel(inter_edge, user_embed, item_embed, Wq, Wk, Wv):
    users = inter_edge[0, :].astype(jnp.int32)
    items = inter_edge[1, :].astype(jnp.int32)
    pad = E_PAD - E
    pad_u = (jnp.arange(pad, dtype=jnp.int32) * 97) % NUM_USERS
    pad_i = (jnp.arange(pad, dtype=jnp.int32) * 89) % NUM_ITEMS
    users_p = jnp.concatenate([users, pad_u])
    items_p = jnp.concatenate([items, pad_i])

    ug, ig = _s0(users_p, items_p, user_embed, item_embed)
    scores3d, bmax = _t2(ug, ig, Wq, Wk)
    gmax = jnp.max(bmax)
    scores = scores3d.reshape(E_PAD)
    gmax_arr = jnp.full((L,), gmax, jnp.float32)
    den, cnt, wg = _s2(users_p, scores, gmax_arr, ig)
    raw = _s3(users_p, wg)
    den3 = den.reshape(NC, U_PAD // 128, 128)
    cnt3 = cnt.reshape(NC, U_PAD // 128, 128)
    out3 = _t4(raw, den3, cnt3, Wv)
    return out3.reshape(U_PAD, EMB)[:NUM_USERS]
